# Initial kernel scaffold; baseline (speedup 1.0000x reference)
import jax
import jax.numpy as jnp
from jax.experimental import pallas as pl


def kernel(input, W_val, row_ids, col_ids):
    raise NotImplementedError("write your pallas kernel here")


# speedup vs baseline: 7.9485x; 7.9485x over previous
import functools

import jax
import jax.numpy as jnp
from jax import lax
from jax.experimental import pallas as pl
from jax.experimental.pallas import tpu as pltpu
from jax.experimental.pallas import tpu_sc as plsc

N = 16384
M = 16384
B = 64
NNZ = 262144

NC = 2
NS = 16
NW = NC * NS
RPT = N // NW
RPC = N // NC
S = 512
QL = 128
Q = S // QL
NNZP = NNZ + S
LANE = 16


def _select(vref, idx):
    v = vref[pl.ds(idx, LANE)]
    return v[0]


mesh = plsc.VectorSubcoreMesh(core_axis_name="c", subcore_axis_name="s")


@functools.partial(
    pl.kernel,
    out_type=jax.ShapeDtypeStruct((N, B), jnp.float32),
    mesh=mesh,
    compiler_params=pltpu.CompilerParams(use_tc_tiling_on_sc=False),
    scratch_types=[
        pltpu.VMEM((64,), jnp.int32),
        pltpu.VMEM((S,), jnp.int32),
        pltpu.VMEM((S,), jnp.int32),
        pltpu.VMEM((S,), jnp.float32),
        pltpu.VMEM((Q, QL), jnp.int32),
        pltpu.VMEM((S, B), jnp.float32),
        pltpu.VMEM_SHARED((RPC, B), jnp.float32),
        pltpu.SemaphoreType.DMA,
    ],
)
def _spmm_sc(xT_hbm, wp_hbm, rowp_hbm, colp_hbm, starts_hbm, out_hbm,
             startsv, colv, rowv, wv, rlv, buf, acc, sem):
    c = lax.axis_index("c")
    s_ax = lax.axis_index("s")
    wid = c * NS + s_ax
    lane16 = jnp.arange(LANE, dtype=jnp.int32)

    def _zero_row(i, _):
        for j4 in range(B // LANE):
            buf[i, pl.ds(j4 * LANE, LANE)] = jnp.zeros((LANE,), jnp.float32)
        return 0
    lax.fori_loop(0, S, _zero_row, 0)
    pltpu.sync_copy(buf, acc.at[pl.ds(s_ax * RPT, RPT)])

    pltpu.sync_copy(starts_hbm, startsv)
    s_lo = _select(startsv, wid)
    s_hi = _select(startsv, wid + 1)
    s_al = (s_lo // QL) * QL
    n_chunks = (s_hi - s_al + S - 1) // S

    def chunk_body(ci, _):
        off = s_al + ci * S
        pltpu.sync_copy(colp_hbm.at[pl.ds(off, S)], colv)
        pltpu.sync_copy(rowp_hbm.at[pl.ds(off, S)], rowv)
        pltpu.sync_copy(wp_hbm.at[pl.ds(off, S)], wv)

        cps = []
        for q in range(Q):
            cps.append(pltpu.async_copy(
                xT_hbm.at[colv.at[pl.ds(q * QL, QL)]],
                buf.at[pl.ds(q * QL, QL)], sem))
        for cp in cps:
            cp.wait()

        for q in range(Q):
            def grp_body(r, _, q=q):
                i0 = q * QL + r * LANE
                w16 = wv[pl.ds(i0, LANE)]
                r16 = rowv[pl.ds(i0, LANE)]
                gidx = off + i0 + lane16
                valid = (gidx >= s_lo) & (gidx < s_hi)
                w16 = jnp.where(valid, w16, jnp.float32(0.0))
                rl16 = jnp.where(valid, r16 - c * RPC, s_ax * RPT)
                rlv[q, pl.ds(r * LANE, LANE)] = rl16

                for k in range(LANE):
                    wk = w16[k]
                    i = i0 + k
                    for j4 in range(B // LANE):
                        sl = pl.ds(j4 * LANE, LANE)
                        buf[i, sl] = buf[i, sl] * wk
                return 0
            lax.fori_loop(0, QL // LANE, grp_body, 0)

            pltpu.sync_copy(buf.at[pl.ds(q * QL, QL)], acc.at[rlv.at[q]],
                            add=True)
        return 0

    lax.fori_loop(0, n_chunks, chunk_body, 0)

    pltpu.sync_copy(acc.at[pl.ds(s_ax * RPT, RPT)],
                    out_hbm.at[pl.ds(wid * RPT, RPT)])


@jax.jit
def kernel(input, W_val, row_ids, col_ids):
    x = input.astype(jnp.float32)
    xT = x.T
    row32 = row_ids.astype(jnp.int32)
    col32 = col_ids.astype(jnp.int32)
    pad = NNZP - NNZ
    wp = jnp.pad(W_val.astype(jnp.float32), (0, pad))
    rowp = jnp.pad(row32, (0, pad), constant_values=N - 1)
    padcols = (jnp.arange(pad, dtype=jnp.int32) * 131) % M
    colp = jnp.concatenate([col32, padcols])
    bounds = jnp.arange(0, N + 1, RPT)
    starts = jnp.searchsorted(row32, bounds.astype(jnp.int32)).astype(jnp.int32)
    starts = jnp.pad(starts, (0, 64 - starts.shape[0]), constant_values=NNZ)
    out_nb = _spmm_sc(xT, wp, rowp, colp, starts)
    return out_nb.T
